# baseline (speedup 1.0000x reference)
import functools

import jax
import jax.numpy as jnp
from jax import lax
from jax.experimental import pallas as pl
from jax.experimental.pallas import tpu as pltpu
from jax.experimental.pallas import tpu_sc as plsc

B, M, C = 4, 25000, 64
Z, X = 512, 512
NC, NS = 2, 16
NW = NC * NS
RPW = Z // NW
L = 16

CH = 2000
NCH = 12
TAIL = 1000
PLIST_CAP = M + 40
GCAP = 64
PEND_CAP = GCAP + 32
FLUSH_AT = 49


def _body(z_hbm, x_hbm, f_hbm, out_hbm,
          zbuf, xbuf, plist, pend, mbuf, fbuf, slab, touched, gsem):
    wid = lax.axis_index("s") * NC + lax.axis_index("c")
    z0 = wid * RPW
    iota = lax.iota(jnp.int32, L)
    zero_f = jnp.zeros((L,), jnp.float32)

    _gdn = lax.GatherDimensionNumbers(
        offset_dims=(), collapsed_slice_dims=(0,), start_index_map=(0,))

    def vperm(v, idx):
        return lax.gather(v, idx[:, None], _gdn, slice_sizes=(1,),
                          mode=lax.GatherScatterMode.PROMISE_IN_BOUNDS)

    def vprefix(m):
        v = jnp.where(m, 1, 0)
        for s in (1, 2, 4, 8):
            sh = vperm(v, jnp.maximum(iota - s, 0))
            v = v + jnp.where(iota >= s, sh, 0)
        return v
    zero_i = jnp.zeros((L,), jnp.int32)
    one_i = jnp.ones((L,), jnp.int32)

    def scan_chunk(ncnt, base_m, nvalid):
        def it(i, ncnt):
            zv = zbuf[pl.ds(i * L, L)]
            xv = xbuf[pl.ds(i * L, L)]
            lanem = (i * L + iota) < nvalid
            zrel = zv - z0
            inr = (zrel >= 0) & (zrel < RPW) & lanem
            psum = vprefix(inr)
            cnt = psum[L - 1]
            gm = base_m + i * L + iota
            p = gm | (xv << 17) | (zrel << 26)
            dest = jnp.where(inr, ncnt + psum - 1, PLIST_CAP - L + iota)
            plsc.store_scatter(plist, [dest], p)
            return ncnt + cnt

        return lax.fori_loop(0, (nvalid + L - 1) // L, it, ncnt)

    def flush(cnt):
        for k in range(GCAP // L):
            pk = pend[pl.ds(k * L, L)]
            valid = (k * L + iota) < cnt
            mbuf[pl.ds(k * L, L)] = jnp.where(valid, pk & 0x1FFFF, 0)
        pltpu.async_copy(f_hbm.at[mbuf], fbuf, gsem).wait()

        def pj_loop(j, _):
            pj = pend[pl.ds(j, L)][0]
            xj = lax.shift_right_logical(pj, 17) & 0x1FF
            xs = jnp.full((L,), xj, jnp.int32)
            tv = plsc.load_gather(touched, [xs])
            first = tv == 0
            for q in range(C // L):
                cvec = q * L + iota
                fv = fbuf[j, pl.ds(q * L, L)]
                cur = plsc.load_gather(slab, [cvec, xs])
                new = jnp.where(first, fv, jnp.maximum(cur, fv))
                plsc.store_scatter(slab, [cvec, xs], new)
            tdest = jnp.where(iota == 0, xs, X + iota)
            plsc.store_scatter(touched, [tdest], one_i)
            return 0

        lax.fori_loop(0, cnt, pj_loop, 0)

    def per_batch(b, _):
        def g_loop(g, n):
            off = b * M + g * CH
            pltpu.sync_copy(z_hbm.at[pl.ds(off, CH)], zbuf.at[pl.ds(0, CH)])
            pltpu.sync_copy(x_hbm.at[pl.ds(off, CH)], xbuf.at[pl.ds(0, CH)])
            return scan_chunk(n, off, CH)

        n = lax.fori_loop(0, NCH, g_loop, 0)
        toff = b * M + NCH * CH
        pltpu.sync_copy(z_hbm.at[pl.ds(toff, TAIL)], zbuf.at[pl.ds(0, TAIL)])
        pltpu.sync_copy(x_hbm.at[pl.ds(toff, TAIL)], xbuf.at[pl.ds(0, TAIL)])
        n = scan_chunk(n, toff, TAIL)

        nch = (n + L - 1) // L

        def row(r, _):
            def zc(c, _):
                for k in range(X // L):
                    slab[c, pl.ds(k * L, L)] = zero_f
                return 0

            lax.fori_loop(0, C, zc, 0)
            for k in range(X // L):
                touched[pl.ds(k * L, L)] = zero_i

            def it(i, pc):
                pv = plist[pl.ds(i * L, L)]
                lanem = (i * L + iota) < n
                zrel = lax.shift_right_logical(pv, 26)
                match = (zrel == r) & lanem
                psum = vprefix(match)
                cnt = psum[L - 1]
                dest = jnp.where(match, pc + psum - 1, PEND_CAP - L + iota)
                plsc.store_scatter(pend, [dest], pv)
                pc = pc + cnt

                @pl.when(pc >= FLUSH_AT)
                def _():
                    flush(pc)

                return jnp.where(pc >= FLUSH_AT, 0, pc)

            pc = lax.fori_loop(0, nch, it, 0)

            @pl.when(pc > 0)
            def _():
                flush(pc)

            pltpu.sync_copy(slab, out_hbm.at[b, :, z0 + r, :])
            return 0

        lax.fori_loop(0, RPW, row, 0)
        return 0

    lax.fori_loop(0, B, per_batch, 0)


_sc_call = functools.partial(
    pl.kernel,
    out_type=jax.ShapeDtypeStruct((B, C, Z, X), jnp.float32),
    mesh=plsc.VectorSubcoreMesh(core_axis_name="c", subcore_axis_name="s"),
    compiler_params=pltpu.CompilerParams(needs_layout_passes=False, use_tc_tiling_on_sc=False),
    scratch_types=[
        pltpu.VMEM((CH,), jnp.int32),
        pltpu.VMEM((CH,), jnp.int32),
        pltpu.VMEM((PLIST_CAP,), jnp.int32),
        pltpu.VMEM((PEND_CAP,), jnp.int32),
        pltpu.VMEM((GCAP,), jnp.int32),
        pltpu.VMEM((GCAP, C), jnp.float32),
        pltpu.VMEM((C, X), jnp.float32),
        pltpu.VMEM((X + L,), jnp.int32),
        pltpu.SemaphoreType.DMA,
    ],
)(_body)


def kernel(voxel_features, voxel_coords):
    z = voxel_coords[:, :, 0].reshape(-1)
    x = voxel_coords[:, :, 2].reshape(-1)
    f = voxel_features.reshape(B * M, C)
    return _sc_call(z, x, f)
